# baseline (speedup 1.0000x reference)
import functools

import jax
import jax.numpy as jnp
from jax import lax
from jax.experimental import pallas as pl
from jax.experimental.pallas import tpu as pltpu
from jax.experimental.pallas import tpu_sc as plsc

_PNUM = 128
_BATCH = 1024
_NCHUNK = _PNUM // 16


def _smooth_l1_sum(p, g, acc):
    d = p - g
    ad = jnp.abs(d)
    m = jnp.minimum(ad, 1.0)
    return acc + m * (ad - 0.5 * m)


def _make_sc_kernel(n_workers, b_per_w):
    mesh = plsc.VectorSubcoreMesh(core_axis_name="c", subcore_axis_name="s")

    @functools.partial(
        pl.kernel,
        mesh=mesh,
        out_type=jax.ShapeDtypeStruct((n_workers, 16), jnp.float32),
        scratch_types=[
            pltpu.VMEM((b_per_w * _PNUM,), jnp.float32),
            pltpu.VMEM((b_per_w * _PNUM,), jnp.float32),
            pltpu.VMEM((b_per_w * 2 * _PNUM,), jnp.float32),
            pltpu.VMEM((b_per_w * 2 * _PNUM,), jnp.float32),
            pltpu.VMEM((16,), jnp.float32),
        ],
        compiler_params=pltpu.CompilerParams(needs_layout_passes=False),
    )
    def sc_kernel(px_hbm, py_hbm, gx_hbm, gy_hbm, out_hbm,
                  px_v, py_v, gx_v, gy_v, out_v):
        nc = 2
        wid = lax.axis_index("s") * nc + lax.axis_index("c")
        base = wid * b_per_w
        pltpu.sync_copy(px_hbm.at[pl.ds(base * _PNUM, b_per_w * _PNUM)], px_v)
        pltpu.sync_copy(py_hbm.at[pl.ds(base * _PNUM, b_per_w * _PNUM)], py_v)
        pltpu.sync_copy(
            gx_hbm.at[pl.ds(base * 2 * _PNUM, b_per_w * 2 * _PNUM)], gx_v)
        pltpu.sync_copy(
            gy_hbm.at[pl.ds(base * 2 * _PNUM, b_per_w * 2 * _PNUM)], gy_v)

        lane = jnp.arange(16, dtype=jnp.int32)

        @plsc.parallel_loop(0, b_per_w, carry=jnp.float32(0.0))
        def bacc(b, bacc):
            px = [px_v[pl.ds(b * _PNUM + c * 16, 16)] for c in range(_NCHUNK)]
            py = [py_v[pl.ds(b * _PNUM + c * 16, 16)] for c in range(_NCHUNK)]
            gbase = b * 2 * _PNUM

            def shift_body(i, smin):
                idx0 = gbase + i + lane
                acc = jnp.zeros((16,), jnp.float32)
                for c in range(_NCHUNK):
                    idx = idx0 + c * 16
                    gx = plsc.load_gather(gx_v, [idx])
                    gy = plsc.load_gather(gy_v, [idx])
                    acc = _smooth_l1_sum(px[c], gx, acc)
                    acc = _smooth_l1_sum(py[c], gy, acc)
                return jnp.minimum(smin, jnp.sum(acc))

            smin = lax.fori_loop(0, _PNUM, shift_body,
                                 jnp.float32(jnp.inf))
            return bacc + smin

        out_v[...] = jnp.zeros((16,), jnp.float32) + bacc
        pltpu.sync_copy(out_v, out_hbm.at[wid])

    return sc_kernel


@jax.jit
def kernel(pred, gt):
    n_workers = 32
    b_per_w = _BATCH // n_workers
    px = pred[:, :, 0].reshape(-1)
    py = pred[:, :, 1].reshape(-1)
    gt2 = jnp.concatenate([gt, gt], axis=1)
    gx = gt2[:, :, 0].reshape(-1)
    gy = gt2[:, :, 1].reshape(-1)
    partials = _make_sc_kernel(n_workers, b_per_w)(px, py, gx, gy)
    return jnp.sum(partials[:, 0]) * (1.0 / (_BATCH * _PNUM))
